# baseline (speedup 1.0000x reference)
import functools
import math

import jax
import jax.numpy as jnp
from jax import lax
from jax.experimental import pallas as pl
from jax.experimental.pallas import tpu as pltpu
from jax.experimental.pallas import tpu_sc as plsc

N = 10000
K = 32
E = N * K
D = 128
DW = D // 2
H = 4
DK = D // H

BN = 200
BE = BN * K
GRID = N // BN
HGRID = GRID // 2

_INV_SQRT_DK = 1.0 / math.sqrt(DK)
_HI_MASK = -65536


def _dot(a, b):
    return jnp.dot(a, b, preferred_element_type=jnp.float32)


def _pack_bf16(x):
    b = lax.bitcast_convert_type(x, jnp.int32) + 0x8000
    lo = (b[:, :DW] >> 16) & 0xffff
    hi = b[:, DW:] & _HI_MASK
    return lo | hi


def _unpack_bf16(p):
    lo = lax.bitcast_convert_type(p << 16, jnp.float32)
    hi = lax.bitcast_convert_type(p & _HI_MASK, jnp.float32)
    return jnp.concatenate([lo, hi], axis=1)


def _pairswap(x):
    r, c = x.shape
    up = jnp.roll(x, -1, axis=0)
    dn = jnp.roll(x, 1, axis=0)
    row = lax.broadcasted_iota(jnp.int32, (r, c), 0)
    return jnp.where(row % 2 == 0, up, dn)


def _head_blockdiag():
    d = lax.broadcasted_iota(jnp.int32, (D, D), 0)
    c = lax.broadcasted_iota(jnp.int32, (D, D), 1)
    return jnp.where(d // DK == c // K, 1.0, 0.0).astype(jnp.float32)


def _segsum_k(x):
    return x.reshape(x.shape[0] // K, K, D).sum(axis=1)


def _mha_residual(fh, mail, Wq, bq, Wk, bk, Wv, bv, Wo, bo):
    q = _dot(fh, Wq) + bq
    k = _dot(mail, Wk) + bk
    v = _dot(mail, Wv) + bv
    qe = jnp.broadcast_to(q[:, None, :], (BN, K, D)).reshape(BE, D)
    s = _dot(qe * k, _head_blockdiag()) * _INV_SQRT_DK
    u = jnp.exp(s)
    numer = _segsum_k(u * v)
    denom = _segsum_k(u)
    o = numer / denom
    return _dot(o, Wo) + bo + fh


def _p1_body(xe_ref, f_ref, Wq_ref, bq_ref, Wk_ref, bk_ref, Wv_ref, bv_ref,
             Wo_ref, bo_ref, fh1_ref, fh1p_ref):
    fh1 = _mha_residual(
        f_ref[...], xe_ref[...],
        Wq_ref[...], bq_ref[...], Wk_ref[...], bk_ref[...],
        Wv_ref[...], bv_ref[...], Wo_ref[...], bo_ref[...])
    fh1_ref[...] = fh1
    fh1p_ref[...] = _pack_bf16(fh1)


def _p2_body(xe_ref, gp_ref, fh1_ref, Wq_ref, bq_ref, Wk_ref, bk_ref,
             Wv_ref, bv_ref, Wo_ref, bo_ref, W0_ref, b0_ref,
             h1p_ref, fh2_ref, fh2p_ref):
    xe = xe_ref[...]
    m = _unpack_bf16(gp_ref[...]) - _pairswap(xe)
    h1 = jnp.maximum(xe + _dot(m, W0_ref[...]) + b0_ref[...], 0.0)
    h1p_ref[...] = _pack_bf16(h1)
    fh2 = _mha_residual(
        fh1_ref[...], h1,
        Wq_ref[...], bq_ref[...], Wk_ref[...], bk_ref[...],
        Wv_ref[...], bv_ref[...], Wo_ref[...], bo_ref[...])
    fh2_ref[...] = fh2
    fh2p_ref[...] = _pack_bf16(fh2)


def _p3_body(xe_ref, gp_ref, h1p_ref, fh2_ref, f_ref, W1_ref, b1_ref,
             Wl_ref, bl_ref, out_ref):
    xe = xe_ref[...]
    m = _unpack_bf16(gp_ref[...]) - _unpack_bf16(_pairswap(h1p_ref[...]))
    h2 = jnp.maximum(xe + _dot(m, W1_ref[...]) + b1_ref[...], 0.0)
    mail_sum = _segsum_k(h2)
    Wl = Wl_ref[...]
    out_ref[...] = (_dot(mail_sum, Wl[0:D]) + _dot(fh2_ref[...], Wl[D:2 * D])
                    + _dot(f_ref[...], Wl[2 * D:3 * D]) + bl_ref[...])


def _edge_spec(off):
    return pl.BlockSpec((BE, D), lambda i, o=off: (i + o, 0))


def _node_spec(off):
    return pl.BlockSpec((BN, D), lambda i, o=off: (i + o, 0))


def _w_spec(rows):
    return pl.BlockSpec((rows, D), lambda i: (0, 0))


def _b_spec():
    return pl.BlockSpec((1, D), lambda i: (0, 0))


def _make_sc_gather(rows_total):
    info = plsc.get_sparse_core_info()
    nw = info.num_cores * info.num_subcores
    per_w = rows_total // nw
    ch = 200
    n_ch = per_w // ch
    pairs = n_ch // 2
    tail = n_ch - 2 * pairs
    mesh = plsc.VectorSubcoreMesh(core_axis_name="c", subcore_axis_name="s")

    @functools.partial(
        pl.kernel,
        out_type=jax.ShapeDtypeStruct((rows_total, DW), jnp.int32),
        mesh=mesh,
        compiler_params=pltpu.CompilerParams(use_tc_tiling_on_sc=False),
        scratch_types=[
            pltpu.VMEM((ch,), jnp.int32),
            pltpu.VMEM((ch,), jnp.int32),
            pltpu.VMEM((ch, DW), jnp.int32),
            pltpu.VMEM((ch, DW), jnp.int32),
            pltpu.SemaphoreType.DMA,
            pltpu.SemaphoreType.DMA,
            pltpu.SemaphoreType.DMA,
            pltpu.SemaphoreType.DMA,
        ],
    )
    def gather(table_hbm, idx_hbm, out_hbm, idx_a, idx_b, rows_a, rows_b,
               gs_a, gs_b, ss_a, ss_b):
        wid = lax.axis_index("s") * info.num_cores + lax.axis_index("c")
        base = wid * per_w
        idx_v = (idx_a, idx_b)
        rows_v = (rows_a, rows_b)
        gs = (gs_a, gs_b)
        ss = (ss_a, ss_b)

        def store_wait(b):
            pltpu.make_async_copy(rows_v[b], out_hbm.at[pl.ds(base, ch)],
                                  ss[b]).wait()

        def body(i, _):
            @pl.when(i > 0)
            def _():
                for b in range(2):
                    store_wait(b)
            handles = []
            for b in range(2):
                off = base + (2 * i + b) * ch
                pltpu.sync_copy(idx_hbm.at[pl.ds(off, ch)], idx_v[b])
                handles.append(
                    pltpu.async_copy(table_hbm.at[idx_v[b]], rows_v[b],
                                     gs[b]))
            for b in range(2):
                off = base + (2 * i + b) * ch
                handles[b].wait()
                pltpu.async_copy(rows_v[b], out_hbm.at[pl.ds(off, ch)],
                                 ss[b])
            return ()

        lax.fori_loop(0, pairs, body, ())
        for b in range(2):
            store_wait(b)
        if tail:
            off = base + 2 * pairs * ch
            pltpu.sync_copy(idx_hbm.at[pl.ds(off, ch)], idx_a)
            pltpu.async_copy(table_hbm.at[idx_a], rows_a, gs_a).wait()
            pltpu.sync_copy(rows_a, out_hbm.at[pl.ds(off, ch)])

    return gather


def kernel(f, x_e, src, Wq, bq, Wk, bk, Wv, bv, Wo, bo, W0, b0, W1, b1,
           Wl, bl):
    bq2, bk2, bv2, bo2, b02, b12, bl2 = (
        b.reshape(1, D) for b in (bq, bk, bv, bo, b0, b1, bl))

    p1 = pl.pallas_call(
        _p1_body,
        grid=(GRID,),
        in_specs=[_edge_spec(0), _node_spec(0),
                  _w_spec(D), _b_spec(), _w_spec(D), _b_spec(),
                  _w_spec(D), _b_spec(), _w_spec(D), _b_spec()],
        out_specs=[pl.BlockSpec((BN, D), lambda i: (i, 0)),
                   pl.BlockSpec((BN, DW), lambda i: (i, 0))],
        out_shape=[jax.ShapeDtypeStruct((N, D), jnp.float32),
                   jax.ShapeDtypeStruct((N, DW), jnp.int32)],
    )
    fh1, fh1p = p1(x_e, f, Wq, bq2, Wk, bk2, Wv, bv2, Wo, bo2)

    sc_gather = _make_sc_gather(E // 2)
    src_a, src_b = src[:E // 2], src[E // 2:]

    def p2_half(half, gp, fh1_full):
        off_e = half * HGRID
        call = pl.pallas_call(
            _p2_body,
            grid=(HGRID,),
            in_specs=[_edge_spec(off_e),
                      pl.BlockSpec((BE, DW), lambda i: (i, 0)),
                      _node_spec(off_e),
                      _w_spec(D), _b_spec(), _w_spec(D), _b_spec(),
                      _w_spec(D), _b_spec(), _w_spec(D), _b_spec(),
                      _w_spec(D), _b_spec()],
            out_specs=[pl.BlockSpec((BE, DW), lambda i: (i, 0)),
                       pl.BlockSpec((BN, D), lambda i: (i, 0)),
                       pl.BlockSpec((BN, DW), lambda i: (i, 0))],
            out_shape=[jax.ShapeDtypeStruct((E // 2, DW), jnp.int32),
                       jax.ShapeDtypeStruct((N // 2, D), jnp.float32),
                       jax.ShapeDtypeStruct((N // 2, DW), jnp.int32)],
        )
        return call(x_e, gp, fh1_full, Wq, bq2, Wk, bk2, Wv, bv2, Wo, bo2,
                    W0, b02)

    def p3_half(half, gp, h1p, fh2):
        off_e = half * HGRID
        call = pl.pallas_call(
            _p3_body,
            grid=(HGRID,),
            in_specs=[_edge_spec(off_e),
                      pl.BlockSpec((BE, DW), lambda i: (i, 0)),
                      pl.BlockSpec((BE, DW), lambda i: (i, 0)),
                      pl.BlockSpec((BN, D), lambda i: (i, 0)),
                      _node_spec(off_e), _w_spec(D), _b_spec(),
                      pl.BlockSpec((3 * D, D), lambda i: (0, 0)), _b_spec()],
            out_specs=pl.BlockSpec((BN, D), lambda i: (i, 0)),
            out_shape=jax.ShapeDtypeStruct((N // 2, D), jnp.float32),
        )
        return call(x_e, gp, h1p, fh2, f, W1, b12, Wl, bl2)

    g0a = sc_gather(fh1p, src_a)
    g0b = sc_gather(fh1p, src_b)
    h1pa, fh2a, fh2pa = p2_half(0, g0a, fh1)
    h1pb, fh2b, fh2pb = p2_half(1, g0b, fh1)
    fh2p = jnp.concatenate([fh2pa, fh2pb], axis=0)

    g1a = sc_gather(fh2p, src_a)
    g1b = sc_gather(fh2p, src_b)
    outa = p3_half(0, g1a, h1pa, fh2a)
    outb = p3_half(1, g1b, h1pb, fh2b)
    return jnp.concatenate([outa, outb], axis=0)


# speedup vs baseline: 1.4039x; 1.4039x over previous
import functools
import math

import jax
import jax.numpy as jnp
from jax import lax
from jax.experimental import pallas as pl
from jax.experimental.pallas import tpu as pltpu
from jax.experimental.pallas import tpu_sc as plsc

N = 10000
K = 32
E = N * K
D = 128
H = 4
DK = D // H

BN = 200
BE = BN * K
GRID = N // BN
S = 5
SGRID = GRID // S
SE = E // S
SN = N // S

_INV_SQRT_DK = 1.0 / math.sqrt(DK)


def _dot(a, b):
    return jnp.dot(a, b, preferred_element_type=jnp.float32)


def _pairswap(x):
    r, c = x.shape
    up = jnp.roll(x, -1, axis=0)
    dn = jnp.roll(x, 1, axis=0)
    row = lax.broadcasted_iota(jnp.int32, (r, c), 0)
    return jnp.where(row % 2 == 0, up, dn)


def _head_blockdiag():
    d = lax.broadcasted_iota(jnp.int32, (D, D), 0)
    c = lax.broadcasted_iota(jnp.int32, (D, D), 1)
    return jnp.where(d // DK == c // K, 1.0, 0.0).astype(jnp.float32)


def _segsum_k(x):
    return x.reshape(x.shape[0] // K, K, D).sum(axis=1)


def _mha_residual(fh, mail, Wq, bq, Wk, bk, Wv, bv, Wo, bo):
    q = _dot(fh, Wq) + bq
    k = _dot(mail, Wk) + bk
    v = _dot(mail, Wv) + bv
    qe = jnp.broadcast_to(q[:, None, :], (BN, K, D)).reshape(BE, D)
    s = _dot(qe * k, _head_blockdiag()) * _INV_SQRT_DK
    u = jnp.exp(s)
    numer = _segsum_k(u * v)
    denom = _segsum_k(u)
    o = numer / denom
    return _dot(o, Wo) + bo + fh


def _p1_body(xe_ref, f_ref, Wq_ref, bq_ref, Wk_ref, bk_ref, Wv_ref, bv_ref,
             Wo_ref, bo_ref, fh1_ref, xeb_ref):
    xe = xe_ref[...]
    fh1_ref[...] = _mha_residual(
        f_ref[...], xe,
        Wq_ref[...], bq_ref[...], Wk_ref[...], bk_ref[...],
        Wv_ref[...], bv_ref[...], Wo_ref[...], bo_ref[...])
    xeb_ref[...] = xe.astype(jnp.bfloat16)


def _p2_body(xeb_ref, g_ref, fh1_ref, Wq_ref, bq_ref, Wk_ref, bk_ref,
             Wv_ref, bv_ref, Wo_ref, bo_ref, W0_ref, b0_ref,
             h1b_ref, fh2_ref):
    xe = xeb_ref[...].astype(jnp.float32)
    m = g_ref[...] - _pairswap(xe)
    h1 = jnp.maximum(xe + _dot(m, W0_ref[...]) + b0_ref[...], 0.0)
    h1b_ref[...] = h1.astype(jnp.bfloat16)
    fh2_ref[...] = _mha_residual(
        fh1_ref[...], h1,
        Wq_ref[...], bq_ref[...], Wk_ref[...], bk_ref[...],
        Wv_ref[...], bv_ref[...], Wo_ref[...], bo_ref[...])


def _p3_body(xeb_ref, g_ref, h1b_ref, fh2_ref, f_ref, W1_ref, b1_ref,
             Wl_ref, bl_ref, out_ref):
    xe = xeb_ref[...].astype(jnp.float32)
    h1 = h1b_ref[...].astype(jnp.float32)
    m = g_ref[...] - _pairswap(h1)
    h2 = jnp.maximum(xe + _dot(m, W1_ref[...]) + b1_ref[...], 0.0)
    mail_sum = _segsum_k(h2)
    Wl = Wl_ref[...]
    out_ref[...] = (_dot(mail_sum, Wl[0:D]) + _dot(fh2_ref[...], Wl[D:2 * D])
                    + _dot(f_ref[...], Wl[2 * D:3 * D]) + bl_ref[...])


def _edge_spec(off):
    return pl.BlockSpec((BE, D), lambda i, o=off: (i + o, 0))


def _node_spec(off):
    return pl.BlockSpec((BN, D), lambda i, o=off: (i + o, 0))


def _w_spec(rows):
    return pl.BlockSpec((rows, D), lambda i: (0, 0))


def _b_spec():
    return pl.BlockSpec((1, D), lambda i: (0, 0))


def _make_sc_gather(rows_total):
    info = plsc.get_sparse_core_info()
    nw = info.num_cores * info.num_subcores
    per_w = rows_total // nw
    ch = 200
    n_ch = per_w // ch
    pairs = n_ch // 2
    tail = n_ch - 2 * pairs
    mesh = plsc.VectorSubcoreMesh(core_axis_name="c", subcore_axis_name="s")

    @functools.partial(
        pl.kernel,
        out_type=jax.ShapeDtypeStruct((rows_total, D), jnp.float32),
        mesh=mesh,
        scratch_types=[
            pltpu.VMEM((ch,), jnp.int32),
            pltpu.VMEM((ch,), jnp.int32),
            pltpu.VMEM((ch, D), jnp.float32),
            pltpu.VMEM((ch, D), jnp.float32),
            pltpu.SemaphoreType.DMA,
            pltpu.SemaphoreType.DMA,
            pltpu.SemaphoreType.DMA,
            pltpu.SemaphoreType.DMA,
        ],
    )
    def gather(table_hbm, idx_hbm, out_hbm, idx_a, idx_b, rows_a, rows_b,
               gs_a, gs_b, ss_a, ss_b):
        wid = lax.axis_index("s") * info.num_cores + lax.axis_index("c")
        base = wid * per_w
        idx_v = (idx_a, idx_b)
        rows_v = (rows_a, rows_b)
        gs = (gs_a, gs_b)
        ss = (ss_a, ss_b)

        def store_wait(b):
            pltpu.make_async_copy(rows_v[b], out_hbm.at[pl.ds(base, ch)],
                                  ss[b]).wait()

        def body(i, _):
            @pl.when(i > 0)
            def _():
                for b in range(2):
                    store_wait(b)
            handles = []
            for b in range(2):
                off = base + (2 * i + b) * ch
                pltpu.sync_copy(idx_hbm.at[pl.ds(off, ch)], idx_v[b])
                handles.append(
                    pltpu.async_copy(table_hbm.at[idx_v[b]], rows_v[b],
                                     gs[b]))
            for b in range(2):
                off = base + (2 * i + b) * ch
                handles[b].wait()
                pltpu.async_copy(rows_v[b], out_hbm.at[pl.ds(off, ch)],
                                 ss[b])
            return ()

        lax.fori_loop(0, pairs, body, ())
        for b in range(2):
            store_wait(b)
        if tail:
            off = base + 2 * pairs * ch
            pltpu.sync_copy(idx_hbm.at[pl.ds(off, ch)], idx_a)
            pltpu.async_copy(table_hbm.at[idx_a], rows_a, gs_a).wait()
            pltpu.sync_copy(rows_a, out_hbm.at[pl.ds(off, ch)])

    return gather


def kernel(f, x_e, src, Wq, bq, Wk, bk, Wv, bv, Wo, bo, W0, b0, W1, b1,
           Wl, bl):
    bq2, bk2, bv2, bo2, b02, b12, bl2 = (
        b.reshape(1, D) for b in (bq, bk, bv, bo, b0, b1, bl))

    p1 = pl.pallas_call(
        _p1_body,
        grid=(GRID,),
        in_specs=[_edge_spec(0), _node_spec(0),
                  _w_spec(D), _b_spec(), _w_spec(D), _b_spec(),
                  _w_spec(D), _b_spec(), _w_spec(D), _b_spec()],
        out_specs=[pl.BlockSpec((BN, D), lambda i: (i, 0)),
                   pl.BlockSpec((BE, D), lambda i: (i, 0))],
        out_shape=[jax.ShapeDtypeStruct((N, D), jnp.float32),
                   jax.ShapeDtypeStruct((E, D), jnp.bfloat16)],
    )
    fh1, xeb = p1(x_e, f, Wq, bq2, Wk, bk2, Wv, bv2, Wo, bo2)

    sc_gather = _make_sc_gather(SE)
    src_s = [src[s * SE:(s + 1) * SE] for s in range(S)]

    def xeb_spec(off):
        return pl.BlockSpec((BE, D), lambda i, o=off: (i + o, 0))

    def p2_slice(s, g):
        off = s * SGRID
        call = pl.pallas_call(
            _p2_body,
            grid=(SGRID,),
            in_specs=[xeb_spec(off),
                      pl.BlockSpec((BE, D), lambda i: (i, 0)),
                      _node_spec(off),
                      _w_spec(D), _b_spec(), _w_spec(D), _b_spec(),
                      _w_spec(D), _b_spec(), _w_spec(D), _b_spec(),
                      _w_spec(D), _b_spec()],
            out_specs=[pl.BlockSpec((BE, D), lambda i: (i, 0)),
                       pl.BlockSpec((BN, D), lambda i: (i, 0))],
            out_shape=[jax.ShapeDtypeStruct((SE, D), jnp.bfloat16),
                       jax.ShapeDtypeStruct((SN, D), jnp.float32)],
        )
        return call(xeb, g, fh1, Wq, bq2, Wk, bk2, Wv, bv2, Wo, bo2,
                    W0, b02)

    def p3_slice(s, g, h1b, fh2s):
        off = s * SGRID
        call = pl.pallas_call(
            _p3_body,
            grid=(SGRID,),
            in_specs=[xeb_spec(off),
                      pl.BlockSpec((BE, D), lambda i: (i, 0)),
                      pl.BlockSpec((BE, D), lambda i: (i, 0)),
                      pl.BlockSpec((BN, D), lambda i: (i, 0)),
                      _node_spec(off), _w_spec(D), _b_spec(),
                      pl.BlockSpec((3 * D, D), lambda i: (0, 0)), _b_spec()],
            out_specs=pl.BlockSpec((BN, D), lambda i: (i, 0)),
            out_shape=jax.ShapeDtypeStruct((SN, D), jnp.float32),
        )
        return call(xeb, g, h1b, fh2s, f, W1, b12, Wl, bl2)

    g0 = [sc_gather(fh1, src_s[s]) for s in range(S)]
    p2_out = [p2_slice(s, g0[s]) for s in range(S)]
    h1b = [o[0] for o in p2_out]
    fh2s = [o[1] for o in p2_out]
    fh2 = jnp.concatenate(fh2s, axis=0)

    g1 = [sc_gather(fh2, src_s[s]) for s in range(S)]
    outs = [p3_slice(s, g1[s], h1b[s], fh2s[s]) for s in range(S)]
    return jnp.concatenate(outs, axis=0)
